# Initial kernel scaffold; baseline (speedup 1.0000x reference)
import jax
import jax.numpy as jnp
from jax.experimental import pallas as pl


def kernel(x, pos, batch, W0_0, b0_0, W0_1, b0_1, W1_0, b1_0, W1_1, b1_1):
    raise NotImplementedError("write your pallas kernel here")


# speedup vs baseline: 9.4751x; 9.4751x over previous
import functools

import jax
import jax.numpy as jnp
from jax import lax
from jax.experimental import pallas as pl
from jax.experimental.pallas import tpu as pltpu
from jax.experimental.pallas import tpu_sc as plsc

NPTS = 10000
NPAD = 10240
NROW = NPAD // 128
DF = 128
NSEL = 2500
NSELP = 2560
R_LIST = (0.2, 0.4)
K_LIST = (32, 64)


def _fps_body(px_ref, py_ref, pz_ref, bf_ref, out_ref, mind_ref):
    rows = lax.broadcasted_iota(jnp.int32, (NROW, 128), 0)
    cols = lax.broadcasted_iota(jnp.int32, (NROW, 128), 1)
    fidx = (rows * 128 + cols).astype(jnp.float32)
    lane = lax.broadcasted_iota(jnp.int32, (1, 128), 1)

    px = px_ref[...]
    py = py_ref[...]
    pz = pz_ref[...]
    bf = bf_ref[...]

    out_ref[...] = jnp.zeros((NSELP, 128), jnp.float32)
    mind_ref[...] = jnp.where(fidx < float(NPTS), jnp.inf, -1.0)

    def write_row(i, lx, ly, lz, bv):
        row = jnp.where(lane == 0, lx,
              jnp.where(lane == 1, ly,
              jnp.where(lane == 2, lz,
              jnp.where(lane == 3, bv, 0.0))))
        out_ref[pl.ds(i, 1), :] = row

    lx0 = px_ref[0, 0]
    ly0 = py_ref[0, 0]
    lz0 = pz_ref[0, 0]
    bv0 = bf_ref[0, 0]
    write_row(0, lx0, ly0, lz0, bv0)

    def body(i, carry):
        lx, ly, lz = carry
        dx = px - lx
        dy = py - ly
        dz = pz - lz
        d = dx * dx + dy * dy + dz * dz
        mind = jnp.minimum(mind_ref[...], d)
        mind_ref[...] = mind
        m = jnp.max(mind)
        sel = jnp.min(jnp.where(mind == m, fidx, 1e9))
        hit = fidx == sel
        nlx = jnp.sum(jnp.where(hit, px, 0.0))
        nly = jnp.sum(jnp.where(hit, py, 0.0))
        nlz = jnp.sum(jnp.where(hit, pz, 0.0))
        nbv = jnp.sum(jnp.where(hit, bf, 0.0))
        write_row(i, nlx, nly, nlz, nbv)
        return (nlx, nly, nlz)

    lax.fori_loop(1, NSEL, body, (lx0, ly0, lz0))


def _run_fps(px, py, pz, bf):
    return pl.pallas_call(
        _fps_body,
        out_shape=jax.ShapeDtypeStruct((NSELP, 128), jnp.float32),
        scratch_shapes=[pltpu.VMEM((NROW, 128), jnp.float32)],
    )(px, py, pz, bf)


def _feats_body(xe_ref, w_ref, b_ref, a0_ref, a1_ref):
    y = jnp.dot(xe_ref[...], w_ref[...], preferred_element_type=jnp.float32)
    y = y + b_ref[0:1, :]
    a0_ref[...] = y[:, :DF]
    a1_ref[...] = y[:, DF:]


def _run_feats(xe, w, b):
    blk = 1024
    return pl.pallas_call(
        _feats_body,
        grid=(NPAD // blk,),
        in_specs=[
            pl.BlockSpec((blk, 256), lambda i: (i, 0)),
            pl.BlockSpec((256, 256), lambda i: (0, 0)),
            pl.BlockSpec((8, 256), lambda i: (0, 0)),
        ],
        out_specs=[
            pl.BlockSpec((blk, DF), lambda i: (i, 0)),
            pl.BlockSpec((blk, DF), lambda i: (i, 0)),
        ],
        out_shape=[
            jax.ShapeDtypeStruct((NPAD, DF), jnp.float32),
            jax.ShapeDtypeStruct((NPAD, DF), jnp.float32),
        ],
    )(xe, w, b)


def _topk_body(pos_ref, p_ref, topi_ref, cnt_ref, d2m_ref, *, r2, k):
    bc = p_ref.shape[0]
    cx = p_ref[:, 0:1]
    cy = p_ref[:, 1:2]
    cz = p_ref[:, 2:3]
    px = pos_ref[0:1, :]
    py = pos_ref[1:2, :]
    pz = pos_ref[2:3, :]
    dx = px - cx
    dy = py - cy
    dz = pz - cz
    d2 = dx * dx + dy * dy + dz * dz
    col = lax.broadcasted_iota(jnp.int32, (bc, NPAD), 1).astype(jnp.float32)
    d2m = jnp.where((d2 <= r2) & (col < float(NPTS)), d2, jnp.inf)
    d2m_ref[...] = d2m

    klane = lax.broadcasted_iota(jnp.int32, (bc, k), 1)

    def body(kk, carry):
        topi, cnt = carry
        v = d2m_ref[...]
        m = jnp.min(v, axis=1, keepdims=True)
        ok = m <= r2
        idxf = jnp.min(jnp.where(v == m, col, 1e9), axis=1, keepdims=True)
        idxf = jnp.where(ok, idxf, 0.0)
        d2m_ref[...] = jnp.where(col == idxf, jnp.inf, v)
        topi = topi + jnp.where(klane == kk, idxf, 0.0)
        cnt = cnt + jnp.where(ok, 1.0, 0.0)
        return (topi, cnt)

    z = jnp.zeros((bc, k), jnp.float32)
    topi, cnt = lax.fori_loop(0, k, body, (z, jnp.zeros((bc, 1), jnp.float32)))
    topi_ref[...] = topi.astype(jnp.int32)
    cnt_ref[...] = jnp.broadcast_to(cnt, (bc, 128))


def _run_topk(pos_pl, p, r, k):
    bc = 128
    body = functools.partial(_topk_body, r2=r * r, k=k)
    return pl.pallas_call(
        body,
        grid=(NSELP // bc,),
        in_specs=[
            pl.BlockSpec((8, NPAD), lambda i: (0, 0)),
            pl.BlockSpec((bc, 128), lambda i: (i, 0)),
        ],
        out_specs=[
            pl.BlockSpec((bc, k), lambda i: (i, 0)),
            pl.BlockSpec((bc, 128), lambda i: (i, 0)),
        ],
        out_shape=[
            jax.ShapeDtypeStruct((NSELP, k), jnp.int32),
            jax.ShapeDtypeStruct((NSELP, 128), jnp.float32),
        ],
        scratch_shapes=[pltpu.VMEM((bc, NPAD), jnp.float32)],
    )(pos_pl, p)


def _sc_gather(a, idx, nedges):
    info = plsc.get_sparse_core_info()
    nw = info.num_cores * info.num_subcores
    b_per_w = nedges // nw
    chunk = 128
    n_chunks = b_per_w // chunk

    @functools.partial(
        pl.kernel,
        out_type=jax.ShapeDtypeStruct((nedges, DF), jnp.float32),
        mesh=plsc.VectorSubcoreMesh(core_axis_name="c", subcore_axis_name="s"),
        scratch_types=[
            pltpu.VMEM((chunk,), jnp.int32),
            pltpu.VMEM((chunk, DF), jnp.float32),
            pltpu.SemaphoreType.DMA,
        ],
    )
    def gk(a_hbm, idx_hbm, out_hbm, idx_v, rows_v, sem):
        wid = lax.axis_index("s") * info.num_cores + lax.axis_index("c")
        base = wid * b_per_w

        def body(i, _):
            off = base + i * chunk
            pltpu.sync_copy(idx_hbm.at[pl.ds(off, chunk)], idx_v)
            pltpu.async_copy(a_hbm.at[idx_v], rows_v, sem).wait()
            pltpu.sync_copy(rows_v, out_hbm.at[pl.ds(off, chunk)])
            return 0

        lax.fori_loop(0, n_chunks, body, 0)

    return gk(a, idx)


def _conv_body(g_ref, cnt_ref, p_ref, wp_ref, w2_ref, b2_ref, out_ref, *, k):
    bc = out_ref.shape[0]
    cx = p_ref[:, 0:1]
    cy = p_ref[:, 1:2]
    cz = p_ref[:, 2:3]
    c = cx * wp_ref[0:1, :] + cy * wp_ref[1:2, :] + cz * wp_ref[2:3, :]
    c3 = lax.broadcast_in_dim(c, (bc, k, DF), (0, 2))
    g = g_ref[...].reshape(bc, k, DF)
    h1 = jnp.maximum(g - c3, 0.0).reshape(bc * k, DF)
    h2 = jnp.dot(h1, w2_ref[...], preferred_element_type=jnp.float32)
    h2 = jnp.maximum(h2 + b2_ref[0:1, :], 0.0).reshape(bc, k, DF)
    kio = lax.broadcasted_iota(jnp.int32, (bc, k, DF), 1).astype(jnp.float32)
    cnt3 = lax.broadcast_in_dim(cnt_ref[:, 0:1], (bc, k, DF), (0, 2))
    h2 = jnp.where(kio < cnt3, h2, 0.0)
    out_ref[...] = jnp.max(h2, axis=1)


def _run_conv(g, val, p, wp, w2, b2, k):
    bc = 1024 // k
    body = functools.partial(_conv_body, k=k)
    return pl.pallas_call(
        body,
        grid=(NSELP // bc,),
        in_specs=[
            pl.BlockSpec((bc * k, DF), lambda i: (i, 0)),
            pl.BlockSpec((bc, 128), lambda i: (i, 0)),
            pl.BlockSpec((bc, 128), lambda i: (i, 0)),
            pl.BlockSpec((8, DF), lambda i: (0, 0)),
            pl.BlockSpec((DF, DF), lambda i: (0, 0)),
            pl.BlockSpec((8, DF), lambda i: (0, 0)),
        ],
        out_specs=pl.BlockSpec((bc, DF), lambda i: (i, 0)),
        out_shape=jax.ShapeDtypeStruct((NSELP, DF), jnp.float32),
    )(g, val, p, wp, w2, b2)


def kernel(x, pos, batch, W0_0, b0_0, W0_1, b0_1, W1_0, b1_0, W1_1, b1_1):
    f32 = jnp.float32
    pos_pad = jnp.pad(pos, ((0, NPAD - NPTS), (0, 0)), constant_values=2.0)
    pos_pl = pos_pad.T
    pos_pl8 = jnp.pad(pos_pl, ((0, 5), (0, 0)))
    px = pos_pl[0].reshape(NROW, 128)
    py = pos_pl[1].reshape(NROW, 128)
    pz = pos_pl[2].reshape(NROW, 128)
    bf = jnp.pad(batch.astype(f32), (0, NPAD - NPTS)).reshape(NROW, 128)

    p = _run_fps(px, py, pz, bf)

    xe = jnp.concatenate([x, pos], axis=1)
    xe = jnp.pad(xe, ((0, NPAD - NPTS), (0, 256 - 131)))
    w = jnp.zeros((256, 256), f32)
    w = w.at[:DF, :DF].set(W0_0[:DF])
    w = w.at[DF:131, :DF].set(W0_0[DF:])
    w = w.at[:DF, DF:].set(W1_0[:DF])
    w = w.at[DF:131, DF:].set(W1_0[DF:])
    b = jnp.zeros((8, 256), f32)
    b = b.at[0, :DF].set(b0_0)
    b = b.at[0, DF:].set(b1_0)
    a0, a1 = _run_feats(xe, w, b)

    outs = []
    for r, k, a, wfull, w2, b2 in (
        (R_LIST[0], K_LIST[0], a0, W0_0, W0_1, b0_1),
        (R_LIST[1], K_LIST[1], a1, W1_0, W1_1, b1_1),
    ):
        topi, cnt = _run_topk(pos_pl8, p, r, k)
        g = _sc_gather(a, topi.reshape(-1), NSELP * k)
        wp = jnp.pad(wfull[DF:], ((0, 5), (0, 0)))
        b2p = jnp.zeros((8, DF), f32).at[0].set(b2)
        outs.append(_run_conv(g, cnt, p, wp, w2, b2p, k))

    out = jnp.concatenate([outs[0][:NSEL], outs[1][:NSEL]], axis=1)
    pos_out = p[:NSEL, :3]
    batch_out = p[:NSEL, 3].astype(jnp.int32)
    return (out, pos_out, batch_out)


# speedup vs baseline: 9.6069x; 1.0139x over previous
import functools

import jax
import jax.numpy as jnp
from jax import lax
from jax.experimental import pallas as pl
from jax.experimental.pallas import tpu as pltpu
from jax.experimental.pallas import tpu_sc as plsc

NPTS = 10000
NPAD = 10240
NROW = NPAD // 128
DF = 128
NSEL = 2500
NSELP = 2560
R_LIST = (0.2, 0.4)
K_LIST = (32, 64)


def _fps_body(px_ref, py_ref, pz_ref, ilv_ref, out_ref):
    rows = lax.broadcasted_iota(jnp.int32, (NROW, 128), 0)
    cols = lax.broadcasted_iota(jnp.int32, (NROW, 128), 1)
    fidx = (rows * 128 + cols).astype(jnp.float32)
    lane4 = lax.broadcasted_iota(jnp.int32, (4, 128), 1)

    px = px_ref[...]
    py = py_ref[...]
    pz = pz_ref[...]

    mind0 = jnp.where(fidx < float(NPTS), jnp.inf, -1.0)
    lx0 = px_ref[0, 0]
    ly0 = py_ref[0, 0]
    lz0 = pz_ref[0, 0]
    bv0 = ilv_ref[3, 0]
    out_ref[0, 0] = lx0
    out_ref[1, 0] = ly0
    out_ref[2, 0] = lz0
    out_ref[3, 0] = bv0

    def body(i, carry):
        mind, lx, ly, lz = carry
        dx = px - lx
        dy = py - ly
        dz = pz - lz
        d = dx * dx + dy * dy + dz * dz
        mind = jnp.minimum(mind, d)
        m = jnp.max(mind)
        sel = jnp.min(jnp.where(mind == m, fidx, 1e9))
        si = sel.astype(jnp.int32)
        r4 = (si // 128) * 4
        c = si % 128
        v4 = ilv_ref[pl.ds(r4, 4), :]
        s = jnp.sum(jnp.where(lane4 == c, v4, 0.0), axis=1, keepdims=True)
        nlx = s[0, 0]
        nly = s[1, 0]
        nlz = s[2, 0]
        out_ref[0, i] = nlx
        out_ref[1, i] = nly
        out_ref[2, i] = nlz
        out_ref[3, i] = s[3, 0]
        return (mind, nlx, nly, nlz)

    lax.fori_loop(1, NSEL, body, (mind0, lx0, ly0, lz0))


def _run_fps(px, py, pz, ilv):
    return pl.pallas_call(
        _fps_body,
        out_specs=pl.BlockSpec(memory_space=pltpu.SMEM),
        out_shape=jax.ShapeDtypeStruct((4, NSEL), jnp.float32),
    )(px, py, pz, ilv)


def _feats_body(xe_ref, w_ref, b_ref, a0_ref, a1_ref):
    y = jnp.dot(xe_ref[...], w_ref[...], preferred_element_type=jnp.float32)
    y = y + b_ref[0:1, :]
    a0_ref[...] = y[:, :DF]
    a1_ref[...] = y[:, DF:]


def _run_feats(xe, w, b):
    blk = 1024
    return pl.pallas_call(
        _feats_body,
        grid=(NPAD // blk,),
        in_specs=[
            pl.BlockSpec((blk, 256), lambda i: (i, 0)),
            pl.BlockSpec((256, 256), lambda i: (0, 0)),
            pl.BlockSpec((8, 256), lambda i: (0, 0)),
        ],
        out_specs=[
            pl.BlockSpec((blk, DF), lambda i: (i, 0)),
            pl.BlockSpec((blk, DF), lambda i: (i, 0)),
        ],
        out_shape=[
            jax.ShapeDtypeStruct((NPAD, DF), jnp.float32),
            jax.ShapeDtypeStruct((NPAD, DF), jnp.float32),
        ],
    )(xe, w, b)


def _topk_body(pos_ref, p_ref, topi_ref, cnt_ref, d2m_ref, *, r2, k):
    bc = p_ref.shape[0]
    cx = p_ref[:, 0:1]
    cy = p_ref[:, 1:2]
    cz = p_ref[:, 2:3]
    px = pos_ref[0:1, :]
    py = pos_ref[1:2, :]
    pz = pos_ref[2:3, :]
    dx = px - cx
    dy = py - cy
    dz = pz - cz
    d2 = dx * dx + dy * dy + dz * dz
    col = lax.broadcasted_iota(jnp.int32, (bc, NPAD), 1).astype(jnp.float32)
    d2m = jnp.where((d2 <= r2) & (col < float(NPTS)), d2, jnp.inf)
    d2m_ref[...] = d2m

    klane = lax.broadcasted_iota(jnp.int32, (bc, k), 1)

    def body(kk, carry):
        topi, cnt = carry
        v = d2m_ref[...]
        m = jnp.min(v, axis=1, keepdims=True)
        ok = m <= r2
        idxf = jnp.min(jnp.where(v == m, col, 1e9), axis=1, keepdims=True)
        idxf = jnp.where(ok, idxf, 0.0)
        d2m_ref[...] = jnp.where(col == idxf, jnp.inf, v)
        topi = topi + jnp.where(klane == kk, idxf, 0.0)
        cnt = cnt + jnp.where(ok, 1.0, 0.0)
        return (topi, cnt)

    z = jnp.zeros((bc, k), jnp.float32)
    topi, cnt = lax.fori_loop(0, k, body, (z, jnp.zeros((bc, 1), jnp.float32)))
    topi_ref[...] = topi.astype(jnp.int32)
    cnt_ref[...] = jnp.broadcast_to(cnt, (bc, 128))


def _run_topk(pos_pl, p, r, k):
    bc = 128
    body = functools.partial(_topk_body, r2=r * r, k=k)
    return pl.pallas_call(
        body,
        grid=(NSELP // bc,),
        in_specs=[
            pl.BlockSpec((8, NPAD), lambda i: (0, 0)),
            pl.BlockSpec((bc, 128), lambda i: (i, 0)),
        ],
        out_specs=[
            pl.BlockSpec((bc, k), lambda i: (i, 0)),
            pl.BlockSpec((bc, 128), lambda i: (i, 0)),
        ],
        out_shape=[
            jax.ShapeDtypeStruct((NSELP, k), jnp.int32),
            jax.ShapeDtypeStruct((NSELP, 128), jnp.float32),
        ],
        scratch_shapes=[pltpu.VMEM((bc, NPAD), jnp.float32)],
    )(pos_pl, p)


def _sc_gather(a, idx, nedges):
    info = plsc.get_sparse_core_info()
    nw = info.num_cores * info.num_subcores
    b_per_w = nedges // nw
    chunk = 128
    n_chunks = b_per_w // chunk

    @functools.partial(
        pl.kernel,
        out_type=jax.ShapeDtypeStruct((nedges, DF), jnp.float32),
        mesh=plsc.VectorSubcoreMesh(core_axis_name="c", subcore_axis_name="s"),
        scratch_types=[
            pltpu.VMEM((chunk,), jnp.int32),
            pltpu.VMEM((chunk, DF), jnp.float32),
            pltpu.SemaphoreType.DMA,
        ],
    )
    def gk(a_hbm, idx_hbm, out_hbm, idx_v, rows_v, sem):
        wid = lax.axis_index("s") * info.num_cores + lax.axis_index("c")
        base = wid * b_per_w

        def body(i, _):
            off = base + i * chunk
            pltpu.sync_copy(idx_hbm.at[pl.ds(off, chunk)], idx_v)
            pltpu.async_copy(a_hbm.at[idx_v], rows_v, sem).wait()
            pltpu.sync_copy(rows_v, out_hbm.at[pl.ds(off, chunk)])
            return 0

        lax.fori_loop(0, n_chunks, body, 0)

    return gk(a, idx)


def _conv_body(g_ref, cnt_ref, p_ref, wp_ref, w2_ref, b2_ref, out_ref, *, k):
    bc = out_ref.shape[0]
    cx = p_ref[:, 0:1]
    cy = p_ref[:, 1:2]
    cz = p_ref[:, 2:3]
    c = cx * wp_ref[0:1, :] + cy * wp_ref[1:2, :] + cz * wp_ref[2:3, :]
    c3 = lax.broadcast_in_dim(c, (bc, k, DF), (0, 2))
    g = g_ref[...].reshape(bc, k, DF)
    h1 = jnp.maximum(g - c3, 0.0).reshape(bc * k, DF)
    h2 = jnp.dot(h1, w2_ref[...], preferred_element_type=jnp.float32)
    h2 = jnp.maximum(h2 + b2_ref[0:1, :], 0.0).reshape(bc, k, DF)
    kio = lax.broadcasted_iota(jnp.int32, (bc, k, DF), 1).astype(jnp.float32)
    cnt3 = lax.broadcast_in_dim(cnt_ref[:, 0:1], (bc, k, DF), (0, 2))
    h2 = jnp.where(kio < cnt3, h2, 0.0)
    out_ref[...] = jnp.max(h2, axis=1)


def _run_conv(g, val, p, wp, w2, b2, k):
    bc = 1024 // k
    body = functools.partial(_conv_body, k=k)
    return pl.pallas_call(
        body,
        grid=(NSELP // bc,),
        in_specs=[
            pl.BlockSpec((bc * k, DF), lambda i: (i, 0)),
            pl.BlockSpec((bc, 128), lambda i: (i, 0)),
            pl.BlockSpec((bc, 128), lambda i: (i, 0)),
            pl.BlockSpec((8, DF), lambda i: (0, 0)),
            pl.BlockSpec((DF, DF), lambda i: (0, 0)),
            pl.BlockSpec((8, DF), lambda i: (0, 0)),
        ],
        out_specs=pl.BlockSpec((bc, DF), lambda i: (i, 0)),
        out_shape=jax.ShapeDtypeStruct((NSELP, DF), jnp.float32),
    )(g, val, p, wp, w2, b2)


def kernel(x, pos, batch, W0_0, b0_0, W0_1, b0_1, W1_0, b1_0, W1_1, b1_1):
    f32 = jnp.float32
    pos_pad = jnp.pad(pos, ((0, NPAD - NPTS), (0, 0)), constant_values=2.0)
    pos_pl = pos_pad.T
    pos_pl8 = jnp.pad(pos_pl, ((0, 5), (0, 0)))
    px = pos_pl[0].reshape(NROW, 128)
    py = pos_pl[1].reshape(NROW, 128)
    pz = pos_pl[2].reshape(NROW, 128)
    bf = jnp.pad(batch.astype(f32), (0, NPAD - NPTS)).reshape(NROW, 128)
    ilv = jnp.stack([px, py, pz, bf], axis=1).reshape(4 * NROW, 128)

    psel = _run_fps(px, py, pz, ilv)
    p = jnp.pad(psel.T, ((0, NSELP - NSEL), (0, 124)))

    xe = jnp.concatenate([x, pos], axis=1)
    xe = jnp.pad(xe, ((0, NPAD - NPTS), (0, 256 - 131)))
    w = jnp.zeros((256, 256), f32)
    w = w.at[:DF, :DF].set(W0_0[:DF])
    w = w.at[DF:131, :DF].set(W0_0[DF:])
    w = w.at[:DF, DF:].set(W1_0[:DF])
    w = w.at[DF:131, DF:].set(W1_0[DF:])
    b = jnp.zeros((8, 256), f32)
    b = b.at[0, :DF].set(b0_0)
    b = b.at[0, DF:].set(b1_0)
    a0, a1 = _run_feats(xe, w, b)

    outs = []
    for r, k, a, wfull, w2, b2 in (
        (R_LIST[0], K_LIST[0], a0, W0_0, W0_1, b0_1),
        (R_LIST[1], K_LIST[1], a1, W1_0, W1_1, b1_1),
    ):
        topi, cnt = _run_topk(pos_pl8, p, r, k)
        g = _sc_gather(a, topi.reshape(-1), NSELP * k)
        wp = jnp.pad(wfull[DF:], ((0, 5), (0, 0)))
        b2p = jnp.zeros((8, DF), f32).at[0].set(b2)
        outs.append(_run_conv(g, cnt, p, wp, w2, b2p, k))

    out = jnp.concatenate([outs[0][:NSEL], outs[1][:NSEL]], axis=1)
    pos_out = psel[:3].T
    batch_out = psel[3].astype(jnp.int32)
    return (out, pos_out, batch_out)


# speedup vs baseline: 9.9202x; 1.0326x over previous
import functools

import jax
import jax.numpy as jnp
from jax import lax
from jax.experimental import pallas as pl
from jax.experimental.pallas import tpu as pltpu
from jax.experimental.pallas import tpu_sc as plsc

NPTS = 10000
NPAD = 10240
NROW = NPAD // 128
DF = 128
NSEL = 2500
NSELP = 2560
R_LIST = (0.2, 0.4)
K_LIST = (32, 64)


def _fps_body(px_ref, py_ref, pz_ref, bf_ref, out_ref):
    rows = lax.broadcasted_iota(jnp.int32, (NROW, 128), 0)
    cols = lax.broadcasted_iota(jnp.int32, (NROW, 128), 1)
    fidx = (rows * 128 + cols).astype(jnp.float32)
    lane = lax.broadcasted_iota(jnp.int32, (1, 128), 1)

    px = px_ref[...]
    py = py_ref[...]
    pz = pz_ref[...]
    bf = bf_ref[...]

    mind0 = jnp.where(fidx < float(NPTS), jnp.inf, -1.0)
    out_ref[...] = jnp.zeros((NSELP, 128), jnp.float32)

    def rmax(v):
        return jnp.max(jnp.max(v, axis=0, keepdims=True), axis=1, keepdims=True)

    def rmin(v):
        return jnp.min(jnp.min(v, axis=0, keepdims=True), axis=1, keepdims=True)

    def write_row(i, vx, vy, vz, vb):
        row = jnp.where(lane == 0, vx,
              jnp.where(lane == 1, vy,
              jnp.where(lane == 2, vz,
              jnp.where(lane == 3, vb, 0.0))))
        out_ref[pl.ds(i, 1), :] = row

    lx0 = px[0:1, 0:1]
    ly0 = py[0:1, 0:1]
    lz0 = pz[0:1, 0:1]
    write_row(0, lx0, ly0, lz0, bf[0:1, 0:1])

    def body(i, carry):
        mind, lx, ly, lz = carry
        dx = px - lx
        dy = py - ly
        dz = pz - lz
        d = dx * dx + dy * dy + dz * dz
        mind = jnp.minimum(mind, d)
        m = rmax(mind)
        sel = rmin(jnp.where(mind == m, fidx, 1e9))
        hit = fidx == sel
        nlx = rmax(jnp.where(hit, px, -1e9))
        nly = rmax(jnp.where(hit, py, -1e9))
        nlz = rmax(jnp.where(hit, pz, -1e9))
        nbv = rmax(jnp.where(hit, bf, -1e9))
        write_row(i, nlx, nly, nlz, nbv)
        return (mind, nlx, nly, nlz)

    lax.fori_loop(1, NSEL, body, (mind0, lx0, ly0, lz0))


def _run_fps(px, py, pz, bf):
    return pl.pallas_call(
        _fps_body,
        out_shape=jax.ShapeDtypeStruct((NSELP, 128), jnp.float32),
    )(px, py, pz, bf)


def _feats_body(xe_ref, w_ref, b_ref, a0_ref, a1_ref):
    y = jnp.dot(xe_ref[...], w_ref[...], preferred_element_type=jnp.float32)
    y = y + b_ref[0:1, :]
    a0_ref[...] = y[:, :DF]
    a1_ref[...] = y[:, DF:]


def _run_feats(xe, w, b):
    blk = 1024
    return pl.pallas_call(
        _feats_body,
        grid=(NPAD // blk,),
        in_specs=[
            pl.BlockSpec((blk, 256), lambda i: (i, 0)),
            pl.BlockSpec((256, 256), lambda i: (0, 0)),
            pl.BlockSpec((8, 256), lambda i: (0, 0)),
        ],
        out_specs=[
            pl.BlockSpec((blk, DF), lambda i: (i, 0)),
            pl.BlockSpec((blk, DF), lambda i: (i, 0)),
        ],
        out_shape=[
            jax.ShapeDtypeStruct((NPAD, DF), jnp.float32),
            jax.ShapeDtypeStruct((NPAD, DF), jnp.float32),
        ],
    )(xe, w, b)


def _topk_body(pos_ref, p_ref, topi_ref, cnt_ref, d2m_ref, *, r2, k):
    bc = p_ref.shape[0]
    cx = p_ref[:, 0:1]
    cy = p_ref[:, 1:2]
    cz = p_ref[:, 2:3]
    px = pos_ref[0:1, :]
    py = pos_ref[1:2, :]
    pz = pos_ref[2:3, :]
    dx = px - cx
    dy = py - cy
    dz = pz - cz
    d2 = dx * dx + dy * dy + dz * dz
    col = lax.broadcasted_iota(jnp.int32, (bc, NPAD), 1).astype(jnp.float32)
    d2m = jnp.where((d2 <= r2) & (col < float(NPTS)), d2, jnp.inf)
    d2m_ref[...] = d2m

    klane = lax.broadcasted_iota(jnp.int32, (bc, k), 1)

    def body(kk, carry):
        topi, cnt = carry
        v = d2m_ref[...]
        m = jnp.min(v, axis=1, keepdims=True)
        ok = m <= r2
        idxf = jnp.min(jnp.where(v == m, col, 1e9), axis=1, keepdims=True)
        idxf = jnp.where(ok, idxf, 0.0)
        d2m_ref[...] = jnp.where(col == idxf, jnp.inf, v)
        topi = topi + jnp.where(klane == kk, idxf, 0.0)
        cnt = cnt + jnp.where(ok, 1.0, 0.0)
        return (topi, cnt)

    z = jnp.zeros((bc, k), jnp.float32)
    topi, cnt = lax.fori_loop(0, k, body, (z, jnp.zeros((bc, 1), jnp.float32)))
    topi_ref[...] = topi.astype(jnp.int32)
    cnt_ref[...] = jnp.broadcast_to(cnt, (bc, 128))


def _run_topk(pos_pl, p, r, k):
    bc = 128
    body = functools.partial(_topk_body, r2=r * r, k=k)
    return pl.pallas_call(
        body,
        grid=(NSELP // bc,),
        in_specs=[
            pl.BlockSpec((8, NPAD), lambda i: (0, 0)),
            pl.BlockSpec((bc, 128), lambda i: (i, 0)),
        ],
        out_specs=[
            pl.BlockSpec((bc, k), lambda i: (i, 0)),
            pl.BlockSpec((bc, 128), lambda i: (i, 0)),
        ],
        out_shape=[
            jax.ShapeDtypeStruct((NSELP, k), jnp.int32),
            jax.ShapeDtypeStruct((NSELP, 128), jnp.float32),
        ],
        scratch_shapes=[pltpu.VMEM((bc, NPAD), jnp.float32)],
    )(pos_pl, p)


def _sc_gather(a, idx, nedges):
    info = plsc.get_sparse_core_info()
    nw = info.num_cores * info.num_subcores
    b_per_w = nedges // nw
    chunk = 128
    n_chunks = b_per_w // chunk

    @functools.partial(
        pl.kernel,
        out_type=jax.ShapeDtypeStruct((nedges, DF), jnp.float32),
        mesh=plsc.VectorSubcoreMesh(core_axis_name="c", subcore_axis_name="s"),
        scratch_types=[
            pltpu.VMEM((chunk,), jnp.int32),
            pltpu.VMEM((chunk, DF), jnp.float32),
            pltpu.SemaphoreType.DMA,
        ],
    )
    def gk(a_hbm, idx_hbm, out_hbm, idx_v, rows_v, sem):
        wid = lax.axis_index("s") * info.num_cores + lax.axis_index("c")
        base = wid * b_per_w

        def body(i, _):
            off = base + i * chunk
            pltpu.sync_copy(idx_hbm.at[pl.ds(off, chunk)], idx_v)
            pltpu.async_copy(a_hbm.at[idx_v], rows_v, sem).wait()
            pltpu.sync_copy(rows_v, out_hbm.at[pl.ds(off, chunk)])
            return 0

        lax.fori_loop(0, n_chunks, body, 0)

    return gk(a, idx)


def _conv_body(g_ref, cnt_ref, p_ref, wp_ref, w2_ref, b2_ref, out_ref, *, k):
    bc = out_ref.shape[0]
    cx = p_ref[:, 0:1]
    cy = p_ref[:, 1:2]
    cz = p_ref[:, 2:3]
    c = cx * wp_ref[0:1, :] + cy * wp_ref[1:2, :] + cz * wp_ref[2:3, :]
    c3 = lax.broadcast_in_dim(c, (bc, k, DF), (0, 2))
    g = g_ref[...].reshape(bc, k, DF)
    h1 = jnp.maximum(g - c3, 0.0).reshape(bc * k, DF)
    h2 = jnp.dot(h1, w2_ref[...], preferred_element_type=jnp.float32)
    h2 = jnp.maximum(h2 + b2_ref[0:1, :], 0.0).reshape(bc, k, DF)
    kio = lax.broadcasted_iota(jnp.int32, (bc, k, DF), 1).astype(jnp.float32)
    cnt3 = lax.broadcast_in_dim(cnt_ref[:, 0:1], (bc, k, DF), (0, 2))
    h2 = jnp.where(kio < cnt3, h2, 0.0)
    out_ref[...] = jnp.max(h2, axis=1)


def _run_conv(g, val, p, wp, w2, b2, k):
    bc = 1024 // k
    body = functools.partial(_conv_body, k=k)
    return pl.pallas_call(
        body,
        grid=(NSELP // bc,),
        in_specs=[
            pl.BlockSpec((bc * k, DF), lambda i: (i, 0)),
            pl.BlockSpec((bc, 128), lambda i: (i, 0)),
            pl.BlockSpec((bc, 128), lambda i: (i, 0)),
            pl.BlockSpec((8, DF), lambda i: (0, 0)),
            pl.BlockSpec((DF, DF), lambda i: (0, 0)),
            pl.BlockSpec((8, DF), lambda i: (0, 0)),
        ],
        out_specs=pl.BlockSpec((bc, DF), lambda i: (i, 0)),
        out_shape=jax.ShapeDtypeStruct((NSELP, DF), jnp.float32),
    )(g, val, p, wp, w2, b2)


def kernel(x, pos, batch, W0_0, b0_0, W0_1, b0_1, W1_0, b1_0, W1_1, b1_1):
    f32 = jnp.float32
    pos_pad = jnp.pad(pos, ((0, NPAD - NPTS), (0, 0)), constant_values=2.0)
    pos_pl = pos_pad.T
    pos_pl8 = jnp.pad(pos_pl, ((0, 5), (0, 0)))
    px = pos_pl[0].reshape(NROW, 128)
    py = pos_pl[1].reshape(NROW, 128)
    pz = pos_pl[2].reshape(NROW, 128)
    bf = jnp.pad(batch.astype(f32), (0, NPAD - NPTS)).reshape(NROW, 128)

    p = _run_fps(px, py, pz, bf)

    xe = jnp.concatenate([x, pos], axis=1)
    xe = jnp.pad(xe, ((0, NPAD - NPTS), (0, 256 - 131)))
    w = jnp.zeros((256, 256), f32)
    w = w.at[:DF, :DF].set(W0_0[:DF])
    w = w.at[DF:131, :DF].set(W0_0[DF:])
    w = w.at[:DF, DF:].set(W1_0[:DF])
    w = w.at[DF:131, DF:].set(W1_0[DF:])
    b = jnp.zeros((8, 256), f32)
    b = b.at[0, :DF].set(b0_0)
    b = b.at[0, DF:].set(b1_0)
    a0, a1 = _run_feats(xe, w, b)

    outs = []
    for r, k, a, wfull, w2, b2 in (
        (R_LIST[0], K_LIST[0], a0, W0_0, W0_1, b0_1),
        (R_LIST[1], K_LIST[1], a1, W1_0, W1_1, b1_1),
    ):
        topi, cnt = _run_topk(pos_pl8, p, r, k)
        g = _sc_gather(a, topi.reshape(-1), NSELP * k)
        wp = jnp.pad(wfull[DF:], ((0, 5), (0, 0)))
        b2p = jnp.zeros((8, DF), f32).at[0].set(b2)
        outs.append(_run_conv(g, cnt, p, wp, w2, b2p, k))

    out = jnp.concatenate([outs[0][:NSEL], outs[1][:NSEL]], axis=1)
    pos_out = p[:NSEL, :3]
    batch_out = p[:NSEL, 3].astype(jnp.int32)
    return (out, pos_out, batch_out)
